# baseline (speedup 1.0000x reference)
import functools

import jax
import jax.numpy as jnp
from jax import lax
from jax.experimental import pallas as pl
from jax.experimental.pallas import tpu as pltpu
from jax.experimental.pallas import tpu_sc as plsc

N = 8388608
T = 8
H = 26
W = 40
BINS = 2 * T * H * W

NC = 2
NS = 16
NW = NC * NS
PER_W = N // NW
C = 4096
N_CHUNKS = PER_W // C
NBUF = 4
L = 16
VPC = C // L
SLICE = BINS // NS


def _hist_body(x_hbm, y_hbm, t_hbm, p_hbm, out_hbm,
               x0, y0, t0, p0, x1, y1, t1, p1,
               x2, y2, t2, p2, x3, y3, t3, p3,
               histv, acc, tmp, slots, sem0, sem1, sem2, sem3):
    cid = lax.axis_index("c")
    sid = lax.axis_index("s")
    wid = sid * NC + cid
    ev_base = wid * PER_W

    zero16 = jnp.zeros((L,), dtype=jnp.float32)
    one16 = jnp.full((L,), 1.0, dtype=jnp.float32)

    def fill_zero(i, carry):
        histv[pl.ds(i * L, L)] = zero16
        return carry

    lax.fori_loop(0, BINS // L, fill_zero, 0)

    def start_loads(base, bufs, sem):
        xr, yr, tr, pr = bufs
        pltpu.async_copy(x_hbm.at[pl.ds(base, C)], xr, sem)
        pltpu.async_copy(y_hbm.at[pl.ds(base, C)], yr, sem)
        pltpu.async_copy(t_hbm.at[pl.ds(base, C)], tr, sem)
        pltpu.async_copy(p_hbm.at[pl.ds(base, C)], pr, sem)

    def wait_loads(bufs, sem):
        xr, yr, tr, pr = bufs
        pltpu.make_async_copy(x_hbm.at[pl.ds(0, C)], xr, sem).wait()
        pltpu.make_async_copy(y_hbm.at[pl.ds(0, C)], yr, sem).wait()
        pltpu.make_async_copy(t_hbm.at[pl.ds(0, C)], tr, sem).wait()
        pltpu.make_async_copy(p_hbm.at[pl.ds(0, C)], pr, sem).wait()

    bufs = [(x0, y0, t0, p0), (x1, y1, t1, p1),
            (x2, y2, t2, p2), (x3, y3, t3, p3)]
    sems = [sem0, sem1, sem2, sem3]

    def accumulate(bufs):
        xr, yr, tr, pr = bufs

        @plsc.parallel_loop(0, VPC, unroll=16)
        def vec_body(i):
            s = pl.ds(i * L, L)
            xs = xr[s]
            ys = yr[s]
            ts = tr[s]
            ps = pr[s]
            xi = xs >> 3
            yi = jnp.minimum(ys >> 3, H - 1)
            ti = (ts * jnp.float32(T)).astype(jnp.int32)
            flat = ((ps << 3) + ti) * (H * W) + yi * W + xi
            plsc.addupdate_scatter(histv, [flat], one16)

    for k in range(NBUF - 1):
        start_loads(ev_base + k * C, bufs[k], sems[k])

    def chunk_group(jj, carry):
        for k in range(NBUF):
            c = NBUF * jj + k
            wait_loads(bufs[k], sems[k])
            accumulate(bufs[k])
            nxt = (k + NBUF - 1) % NBUF

            @pl.when(c + NBUF - 1 < N_CHUNKS)
            def _():
                start_loads(ev_base + (c + NBUF - 1) * C,
                            bufs[nxt], sems[nxt])

        return carry

    lax.fori_loop(0, N_CHUNKS // NBUF, chunk_group, 0)

    pltpu.sync_copy(histv, slots.at[pl.ds(sid * BINS, BINS)])
    plsc.subcore_barrier()

    off = sid * SLICE
    pltpu.sync_copy(slots.at[pl.ds(off, SLICE)], acc)

    def red_body(k, carry):
        pltpu.sync_copy(slots.at[pl.ds(k * BINS + off, SLICE)], tmp)

        def add_body(i, carry2):
            s = pl.ds(i * L, L)
            acc[s] = acc[s] + tmp[s]
            return carry2

        lax.fori_loop(0, SLICE // L, add_body, 0)
        return carry

    lax.fori_loop(1, NS, red_body, 0)
    pltpu.sync_copy(acc, out_hbm.at[pl.ds(cid * BINS + off, SLICE)])


@jax.jit
def _voxel_hist(x, y, t, p):
    mesh = plsc.VectorSubcoreMesh(
        core_axis_name="c", subcore_axis_name="s",
        num_cores=NC, num_subcores=NS,
    )
    partials = pl.kernel(
        _hist_body,
        out_type=jax.ShapeDtypeStruct((NC * BINS,), jnp.float32),
        mesh=mesh,
        compiler_params=pltpu.CompilerParams(needs_layout_passes=False),
        scratch_types=(
            [pltpu.VMEM((C,), dt)
             for _ in range(NBUF)
             for dt in (jnp.int32, jnp.int32, jnp.float32, jnp.int32)]
            + [
                pltpu.VMEM((BINS,), jnp.float32),
                pltpu.VMEM((SLICE,), jnp.float32),
                pltpu.VMEM((SLICE,), jnp.float32),
                pltpu.VMEM_SHARED((NS * BINS,), jnp.float32),
            ]
            + [pltpu.SemaphoreType.DMA] * NBUF
        ),
    )(x, y, t, p)
    return partials.reshape(NC, BINS).sum(axis=0).reshape(2 * T, H, W)


def kernel(x, y, t, p):
    return _voxel_hist(x, y, t, p)


# speedup vs baseline: 1.0582x; 1.0582x over previous
import functools

import jax
import jax.numpy as jnp
from jax import lax
from jax.experimental import pallas as pl
from jax.experimental.pallas import tpu as pltpu
from jax.experimental.pallas import tpu_sc as plsc

N = 8388608
T = 8
H = 26
W = 40
BINS = 2 * T * H * W

NC = 2
NS = 16
NW = NC * NS
PER_W = N // NW
C = 4096
N_CHUNKS = PER_W // C
NBUF = 4
L = 16
VPC = C // L
SLICE = BINS // NS


def _hist_body(x_hbm, y_hbm, t_hbm, p_hbm, out_hbm, scr_hbm,
               x0, y0, t0, p0, x1, y1, t1, p1,
               x2, y2, t2, p2, x3, y3, t3, p3,
               histv, acc, tmp, sem0, sem1, sem2, sem3, sem_r):
    cid = lax.axis_index("c")
    sid = lax.axis_index("s")
    wid = sid * NC + cid
    ev_base = wid * PER_W

    zero16 = jnp.zeros((L,), dtype=jnp.float32)
    one16 = jnp.full((L,), 1.0, dtype=jnp.float32)

    def fill_zero(i, carry):
        histv[pl.ds(i * L, L)] = zero16
        return carry

    lax.fori_loop(0, BINS // L, fill_zero, 0)

    def start_loads(base, bufs, sem):
        xr, yr, tr, pr = bufs
        pltpu.async_copy(x_hbm.at[pl.ds(base, C)], xr, sem)
        pltpu.async_copy(y_hbm.at[pl.ds(base, C)], yr, sem)
        pltpu.async_copy(t_hbm.at[pl.ds(base, C)], tr, sem)
        pltpu.async_copy(p_hbm.at[pl.ds(base, C)], pr, sem)

    def wait_loads(bufs, sem):
        xr, yr, tr, pr = bufs
        pltpu.make_async_copy(x_hbm.at[pl.ds(0, C)], xr, sem).wait()
        pltpu.make_async_copy(y_hbm.at[pl.ds(0, C)], yr, sem).wait()
        pltpu.make_async_copy(t_hbm.at[pl.ds(0, C)], tr, sem).wait()
        pltpu.make_async_copy(p_hbm.at[pl.ds(0, C)], pr, sem).wait()

    bufs = [(x0, y0, t0, p0), (x1, y1, t1, p1),
            (x2, y2, t2, p2), (x3, y3, t3, p3)]
    sems = [sem0, sem1, sem2, sem3]

    def accumulate(bufs):
        xr, yr, tr, pr = bufs

        @plsc.parallel_loop(0, VPC, unroll=8)
        def vec_body(i):
            s = pl.ds(i * L, L)
            xs = xr[s]
            ys = yr[s]
            ts = tr[s]
            ps = pr[s]
            xi = xs >> 3
            yi = jnp.minimum(ys >> 3, H - 1)
            ti = (ts * jnp.float32(T)).astype(jnp.int32)
            flat = ((ps << 3) + ti) * (H * W) + yi * W + xi
            plsc.addupdate_scatter(histv, [flat], one16)

    for k in range(NBUF - 1):
        start_loads(ev_base + k * C, bufs[k], sems[k])

    def chunk_group(jj, carry):
        for k in range(NBUF):
            c = NBUF * jj + k
            wait_loads(bufs[k], sems[k])
            accumulate(bufs[k])
            nxt = (k + NBUF - 1) % NBUF

            @pl.when(c + NBUF - 1 < N_CHUNKS)
            def _():
                start_loads(ev_base + (c + NBUF - 1) * C,
                            bufs[nxt], sems[nxt])

        return carry

    lax.fori_loop(0, N_CHUNKS // NBUF, chunk_group, 0)

    pltpu.sync_copy(histv, scr_hbm.at[pl.ds((cid * NS + sid) * BINS, BINS)])
    plsc.subcore_barrier()

    off = sid * SLICE
    descs = []
    for k in range(NS):
        descs.append(pltpu.async_copy(
            scr_hbm.at[pl.ds((cid * NS + k) * BINS + off, SLICE)],
            tmp.at[pl.ds(k * SLICE, SLICE)],
            sem_r,
        ))
    for d in descs:
        d.wait()

    def add_body(i, carry):
        s = i * L
        v = tmp[pl.ds(s, L)]
        for k in range(1, NS):
            v = v + tmp[pl.ds(k * SLICE + s, L)]
        acc[pl.ds(s, L)] = v
        return carry

    lax.fori_loop(0, SLICE // L, add_body, 0)
    pltpu.sync_copy(acc, out_hbm.at[pl.ds(cid * BINS + off, SLICE)])


@jax.jit
def _voxel_hist(x, y, t, p):
    mesh = plsc.VectorSubcoreMesh(
        core_axis_name="c", subcore_axis_name="s",
        num_cores=NC, num_subcores=NS,
    )
    partials, _ = pl.kernel(
        _hist_body,
        out_type=(
            jax.ShapeDtypeStruct((NC * BINS,), jnp.float32),
            jax.ShapeDtypeStruct((NW * BINS,), jnp.float32),
        ),
        mesh=mesh,
        compiler_params=pltpu.CompilerParams(needs_layout_passes=False),
        scratch_types=(
            [pltpu.VMEM((C,), dt)
             for _ in range(NBUF)
             for dt in (jnp.int32, jnp.int32, jnp.float32, jnp.int32)]
            + [
                pltpu.VMEM((BINS,), jnp.float32),
                pltpu.VMEM((SLICE,), jnp.float32),
                pltpu.VMEM((NS * SLICE,), jnp.float32),
            ]
            + [pltpu.SemaphoreType.DMA] * (NBUF + 1)
        ),
    )(x, y, t, p)
    return partials.reshape(NC, BINS).sum(axis=0).reshape(2 * T, H, W)


def kernel(x, y, t, p):
    return _voxel_hist(x, y, t, p)
